# baseline (speedup 1.0000x reference)
import functools

import jax
import jax.numpy as jnp
from jax import lax
from jax.experimental import pallas as pl
from jax.experimental.pallas import tpu as pltpu
from jax.experimental.pallas import tpu_sc as plsc

_RATIO = 0.05


def _fps_body(n_chunks, width, c_dim, n_samples, flat3_ref, flate_ref,
              sel_ref, md_ref, d2_ref, idx_ref, col_ref):
    subl = lax.broadcasted_iota(jnp.int32, (8, width), 0)
    lane = lax.broadcasted_iota(jnp.int32, (8, width), 1)
    md_ref[...] = jnp.where(subl < n_chunks, jnp.inf, -jnp.inf)
    d2_ref[...] = jnp.zeros((8, width), jnp.float32)
    idx_ref[...] = subl * width + lane
    sel_ref[0] = 0
    n_points = n_chunks * width

    def outer(i, idx):
        tile_of = idx // 128
        lane_of = idx % 128
        lanes0 = lax.broadcasted_iota(jnp.int32, (1, 128), 1)
        mask = lanes0 == lane_of
        tile = flate_ref[tile_of]
        col_ref[...] = jnp.sum(jnp.where(mask, tile, 0.0), axis=1,
                               keepdims=True)

        def block(k, bi):
            acc = None
            for r in range(16):
                rr = bi * 16 + r
                x = flat3_ref[k, rr * 8:rr * 8 + 8, :]
                cseg = col_ref[rr * 8:rr * 8 + 8, :]
                dd = x - cseg
                sq = dd * dd
                acc = sq if acc is None else acc + sq
            t1 = acc + pltpu.roll(acc, 4, axis=0)
            t2 = t1 + pltpu.roll(t1, 2, axis=0)
            return t2 + pltpu.roll(t2, 1, axis=0)

        for k in range(n_chunks):
            d28 = (block(k, 0) + block(k, 1)) + block(k, 2)
            d2_ref[k:k + 1, :] = d28[0:1, :]

        mdall = jnp.minimum(md_ref[...], jnp.sqrt(d2_ref[...]))
        md_ref[...] = mdall
        cmax = jnp.max(mdall)
        cand = jnp.where(mdall == cmax, idx_ref[...], n_points)
        bidx = jnp.min(cand)
        sel_ref[i] = bidx
        return bidx

    lax.fori_loop(1, n_samples, outer, jnp.int32(0))


def _fps_select(flat3, flate, n_chunks, width, c_dim, n_samples):
    body = functools.partial(_fps_body, n_chunks, width, c_dim, n_samples)
    return pl.pallas_call(
        body,
        out_shape=jax.ShapeDtypeStruct((n_samples,), jnp.int32),
        in_specs=[pl.BlockSpec(memory_space=pltpu.VMEM),
                  pl.BlockSpec(memory_space=pltpu.VMEM)],
        out_specs=pl.BlockSpec(memory_space=pltpu.SMEM),
        scratch_shapes=[
            pltpu.VMEM((8, width), jnp.float32),
            pltpu.VMEM((8, width), jnp.float32),
            pltpu.VMEM((8, width), jnp.int32),
            pltpu.VMEM((c_dim, 1), jnp.float32),
        ],
    )(flat3, flate)


def _sc_gather(table, idx_padded, b_padded, d_dim):
    info = plsc.get_sparse_core_info()
    num_workers = info.num_cores * info.num_subcores
    b_per_w = b_padded // num_workers
    mesh = plsc.VectorSubcoreMesh(core_axis_name="c", subcore_axis_name="s")

    @functools.partial(
        pl.kernel,
        mesh=mesh,
        out_type=jax.ShapeDtypeStruct((b_padded, d_dim), jnp.float32),
        scratch_types=[
            pltpu.VMEM((b_per_w,), jnp.int32),
            pltpu.VMEM((b_per_w, d_dim), jnp.float32),
            pltpu.SemaphoreType.DMA,
        ],
    )
    def gather_kernel(table_hbm, idx_hbm, out_hbm, idx_v, rows_v, sem):
        wid = lax.axis_index("s") * info.num_cores + lax.axis_index("c")
        base = wid * b_per_w
        pltpu.sync_copy(idx_hbm.at[pl.ds(base, b_per_w)], idx_v)
        pltpu.async_copy(table_hbm.at[idx_v], rows_v, sem).wait()
        pltpu.sync_copy(rows_v, out_hbm.at[pl.ds(base, b_per_w)])

    return gather_kernel(table, idx_padded)


def kernel(features):
    b, c, h, w = features.shape
    flat = jnp.transpose(features, (0, 2, 3, 1)).reshape(b * h * w, c)
    n = flat.shape[0]
    n_samples = max(1, int(n * _RATIO))

    width = 1792
    n_chunks = n // width
    flat_t = flat.T
    flat3 = jnp.transpose(flat_t.reshape(c, n_chunks, width), (1, 0, 2))
    flate = jnp.transpose(flat_t.reshape(c, n // 128, 128), (1, 0, 2))

    sel = _fps_select(flat3, flate, n_chunks, width, c, n_samples)

    b_padded = ((n_samples + 255) // 256) * 256
    idx_padded = jnp.concatenate(
        [sel, jnp.zeros((b_padded - n_samples,), jnp.int32)])
    gathered = _sc_gather(flat, idx_padded, b_padded, c)
    return gathered[:n_samples]


# speedup vs baseline: 1.1110x; 1.1110x over previous
import functools

import jax
import jax.numpy as jnp
from jax import lax
from jax.experimental import pallas as pl
from jax.experimental.pallas import tpu as pltpu
from jax.experimental.pallas import tpu_sc as plsc

_RATIO = 0.05


def _fps_body(n_chunks, width, c_dim, n_samples, flat3_ref, flate_ref,
              sel_ref, md_ref, d2_ref, idx_ref):
    subl = lax.broadcasted_iota(jnp.int32, (8, width), 0)
    lane = lax.broadcasted_iota(jnp.int32, (8, width), 1)
    md_ref[...] = jnp.where(subl < n_chunks, jnp.inf, -jnp.inf)
    d2_ref[...] = jnp.zeros((8, width), jnp.float32)
    idx_ref[...] = subl * width + lane
    sel_ref[0] = 0
    n_points = n_chunks * width

    def outer(i, idx):
        tile_of = idx // 128
        lane_of = idx % 128
        lanes0 = lax.broadcasted_iota(jnp.int32, (1, 128), 1)
        mask = lanes0 == lane_of
        tile = flate_ref[tile_of]
        col = jnp.sum(jnp.where(mask, tile, 0.0), axis=1, keepdims=True)

        def block(k, bi):
            acc = None
            for r in range(16):
                rr = bi * 16 + r
                x = flat3_ref[k, rr * 8:rr * 8 + 8, :]
                cseg = col[rr * 8:rr * 8 + 8, :]
                dd = x - cseg
                sq = dd * dd
                acc = sq if acc is None else acc + sq
            t1 = acc + pltpu.roll(acc, 4, axis=0)
            t2 = t1 + pltpu.roll(t1, 2, axis=0)
            return t2 + pltpu.roll(t2, 1, axis=0)

        for k in range(n_chunks):
            d28 = (block(k, 0) + block(k, 1)) + block(k, 2)
            d2_ref[k:k + 1, :] = d28[0:1, :]

        mdall = jnp.minimum(md_ref[...], jnp.sqrt(d2_ref[...]))
        md_ref[...] = mdall
        cmax = jnp.max(mdall)
        cand = jnp.where(mdall == cmax, idx_ref[...], n_points)
        bidx = jnp.min(cand)
        sel_ref[i] = bidx
        return bidx

    lax.fori_loop(1, n_samples, outer, jnp.int32(0))


def _fps_select(flat3, flate, n_chunks, width, c_dim, n_samples):
    body = functools.partial(_fps_body, n_chunks, width, c_dim, n_samples)
    return pl.pallas_call(
        body,
        out_shape=jax.ShapeDtypeStruct((n_samples,), jnp.int32),
        in_specs=[pl.BlockSpec(memory_space=pltpu.VMEM),
                  pl.BlockSpec(memory_space=pltpu.VMEM)],
        out_specs=pl.BlockSpec(memory_space=pltpu.SMEM),
        scratch_shapes=[
            pltpu.VMEM((8, width), jnp.float32),
            pltpu.VMEM((8, width), jnp.float32),
            pltpu.VMEM((8, width), jnp.int32),
        ],
    )(flat3, flate)


def _sc_gather(table, idx_padded, b_padded, d_dim):
    info = plsc.get_sparse_core_info()
    num_workers = info.num_cores * info.num_subcores
    b_per_w = b_padded // num_workers
    mesh = plsc.VectorSubcoreMesh(core_axis_name="c", subcore_axis_name="s")

    @functools.partial(
        pl.kernel,
        mesh=mesh,
        out_type=jax.ShapeDtypeStruct((b_padded, d_dim), jnp.float32),
        scratch_types=[
            pltpu.VMEM((b_per_w,), jnp.int32),
            pltpu.VMEM((b_per_w, d_dim), jnp.float32),
            pltpu.SemaphoreType.DMA,
        ],
    )
    def gather_kernel(table_hbm, idx_hbm, out_hbm, idx_v, rows_v, sem):
        wid = lax.axis_index("s") * info.num_cores + lax.axis_index("c")
        base = wid * b_per_w
        pltpu.sync_copy(idx_hbm.at[pl.ds(base, b_per_w)], idx_v)
        pltpu.async_copy(table_hbm.at[idx_v], rows_v, sem).wait()
        pltpu.sync_copy(rows_v, out_hbm.at[pl.ds(base, b_per_w)])

    return gather_kernel(table, idx_padded)


def kernel(features):
    b, c, h, w = features.shape
    flat = jnp.transpose(features, (0, 2, 3, 1)).reshape(b * h * w, c)
    n = flat.shape[0]
    n_samples = max(1, int(n * _RATIO))

    width = 1792
    n_chunks = n // width
    flat_t = flat.T
    flat3 = jnp.transpose(flat_t.reshape(c, n_chunks, width), (1, 0, 2))
    flate = jnp.transpose(flat_t.reshape(c, n // 128, 128), (1, 0, 2))

    sel = _fps_select(flat3, flate, n_chunks, width, c, n_samples)

    b_padded = ((n_samples + 255) // 256) * 256
    idx_padded = jnp.concatenate(
        [sel, jnp.zeros((b_padded - n_samples,), jnp.int32)])
    gathered = _sc_gather(flat, idx_padded, b_padded, c)
    return gathered[:n_samples]
